# baseline (speedup 1.0000x reference)
import jax
import jax.numpy as jnp
from jax.experimental import pallas as pl


def _pass_a(x0in_ref, inc_ref, wt_ref, b_ref, xlin_ref, x1_ref):
    @pl.when(pl.program_id(0) == 0)
    def _():
        xlin_ref[...] = (
            jnp.dot(x0in_ref[...], wt_ref[...], preferred_element_type=jnp.float32)
            + b_ref[...]
        )
    x1_ref[...] = jax.lax.dot_general(
        inc_ref[...].astype(jnp.bfloat16),
        xlin_ref[...].astype(jnp.bfloat16),
        dimension_numbers=(((0,), (0,)), ((), ())),
        preferred_element_type=jnp.float32,
    )


def _pass_b(inc_ref, x1_ref, xlin_ref, out_ref):
    acc = jnp.dot(
        inc_ref[...].astype(jnp.bfloat16),
        x1_ref[...].astype(jnp.bfloat16),
        preferred_element_type=jnp.float32,
    )
    ns = jnp.sum(inc_ref[...], axis=1, keepdims=True)
    out_ref[...] = xlin_ref[...] + acc / ns


def kernel(x_0, incidence_1, W, b):
    n, c_in = x_0.shape
    e = incidence_1.shape[1]
    c_hid = W.shape[0]
    wt = W.T
    b2 = b.reshape(1, c_hid)

    be = min(512, e)
    xlin, x_1 = pl.pallas_call(
        _pass_a,
        grid=(pl.cdiv(e, be),),
        in_specs=[
            pl.BlockSpec((n, c_in), lambda i: (0, 0)),
            pl.BlockSpec((n, be), lambda i: (0, i)),
            pl.BlockSpec((c_in, c_hid), lambda i: (0, 0)),
            pl.BlockSpec((1, c_hid), lambda i: (0, 0)),
        ],
        out_specs=[
            pl.BlockSpec((n, c_hid), lambda i: (0, 0)),
            pl.BlockSpec((be, c_hid), lambda i: (i, 0)),
        ],
        out_shape=[
            jax.ShapeDtypeStruct((n, c_hid), jnp.float32),
            jax.ShapeDtypeStruct((e, c_hid), jnp.float32),
        ],
    )(x_0, incidence_1, wt, b2)

    bn = min(512, n)
    x0_out = pl.pallas_call(
        _pass_b,
        grid=(pl.cdiv(n, bn),),
        in_specs=[
            pl.BlockSpec((bn, e), lambda i: (i, 0)),
            pl.BlockSpec((e, c_hid), lambda i: (0, 0)),
            pl.BlockSpec((bn, c_hid), lambda i: (i, 0)),
        ],
        out_specs=pl.BlockSpec((bn, c_hid), lambda i: (i, 0)),
        out_shape=jax.ShapeDtypeStruct((n, c_hid), jnp.float32),
    )(incidence_1, x_1, xlin)

    return (x0_out, x_1)


# speedup vs baseline: 1.0015x; 1.0015x over previous
import jax
import jax.numpy as jnp
from jax.experimental import pallas as pl
from jax.experimental.pallas import tpu as pltpu


def _pass_a(x0in_ref, inc_ref, wt_ref, b_ref, xlin_ref, x1_ref, xlint_ref):
    @pl.when(pl.program_id(0) == 0)
    def _():
        xlin = (
            jnp.dot(x0in_ref[...], wt_ref[...], preferred_element_type=jnp.float32)
            + b_ref[...]
        )
        xlin_ref[...] = xlin
        xlint_ref[...] = xlin.T
    x1t = jnp.dot(xlint_ref[...], inc_ref[...], preferred_element_type=jnp.float32)
    x1_ref[...] = x1t.T


def _pass_b(inc_ref, x1_ref, xlin_ref, out_ref):
    acc = jnp.dot(inc_ref[...], x1_ref[...], preferred_element_type=jnp.float32)
    ns = jnp.sum(inc_ref[...], axis=1, keepdims=True)
    out_ref[...] = xlin_ref[...] + acc / ns


def kernel(x_0, incidence_1, W, b):
    n, c_in = x_0.shape
    e = incidence_1.shape[1]
    c_hid = W.shape[0]
    wt = W.T
    b2 = b.reshape(1, c_hid)

    be = min(512, e)
    xlin, x_1 = pl.pallas_call(
        _pass_a,
        grid=(pl.cdiv(e, be),),
        in_specs=[
            pl.BlockSpec((n, c_in), lambda i: (0, 0)),
            pl.BlockSpec((n, be), lambda i: (0, i)),
            pl.BlockSpec((c_in, c_hid), lambda i: (0, 0)),
            pl.BlockSpec((1, c_hid), lambda i: (0, 0)),
        ],
        out_specs=[
            pl.BlockSpec((n, c_hid), lambda i: (0, 0)),
            pl.BlockSpec((be, c_hid), lambda i: (i, 0)),
        ],
        out_shape=[
            jax.ShapeDtypeStruct((n, c_hid), jnp.float32),
            jax.ShapeDtypeStruct((e, c_hid), jnp.float32),
        ],
        scratch_shapes=[pltpu.VMEM((c_hid, n), jnp.float32)],
    )(x_0, incidence_1, wt, b2)

    bn = min(512, n)
    x0_out = pl.pallas_call(
        _pass_b,
        grid=(pl.cdiv(n, bn),),
        in_specs=[
            pl.BlockSpec((bn, e), lambda i: (i, 0)),
            pl.BlockSpec((e, c_hid), lambda i: (0, 0)),
            pl.BlockSpec((bn, c_hid), lambda i: (i, 0)),
        ],
        out_specs=pl.BlockSpec((bn, c_hid), lambda i: (i, 0)),
        out_shape=jax.ShapeDtypeStruct((n, c_hid), jnp.float32),
    )(incidence_1, x_1, xlin)

    return (x0_out, x_1)


# speedup vs baseline: 1.0521x; 1.0505x over previous
import functools

import jax
import jax.numpy as jnp
from jax.experimental import pallas as pl
from jax.experimental.pallas import tpu as pltpu

_SCALE = 127.0


def _pass_a(x0in_ref, inc_ref, wt_ref, b_ref, xlin_ref, x1_ref, inc8_ref, ns_ref,
            *, e_total):
    @pl.when(pl.program_id(0) == 0)
    def _():
        xlin_ref[...] = (
            jnp.dot(x0in_ref[...], wt_ref[...], preferred_element_type=jnp.float32)
            + b_ref[...]
        )
        ns_ref[...] = jnp.zeros_like(ns_ref)
    blk = inc_ref[...]
    x1_ref[...] = jax.lax.dot_general(
        blk, xlin_ref[...],
        dimension_numbers=(((0,), (0,)), ((), ())),
        preferred_element_type=jnp.float32,
    )
    inc8_ref[...] = (blk * _SCALE + 0.5).astype(jnp.int8)
    col = (jax.lax.broadcasted_iota(jnp.int32, blk.shape, 1)
           + pl.program_id(0) * blk.shape[1])
    ns_ref[...] += jnp.sum(jnp.where(col < e_total, blk, 0.0), axis=1,
                           keepdims=True)


def _pass_b(inc8_ref, x1_ref, xlin_ref, ns_ref, out_ref, x1bf_ref):
    @pl.when(pl.program_id(0) == 0)
    def _():
        x1bf_ref[...] = x1_ref[...].astype(jnp.bfloat16)
    acc = jnp.dot(
        inc8_ref[...].astype(jnp.bfloat16), x1bf_ref[...],
        preferred_element_type=jnp.float32,
    )
    out_ref[...] = xlin_ref[...] + acc * (1.0 / _SCALE) / ns_ref[...]


def kernel(x_0, incidence_1, W, b):
    n, c_in = x_0.shape
    e = incidence_1.shape[1]
    c_hid = W.shape[0]
    wt = W.T
    b2 = b.reshape(1, c_hid)

    be = min(384, e)
    xlin, x_1, inc8, ns = pl.pallas_call(
        functools.partial(_pass_a, e_total=e),
        grid=(pl.cdiv(e, be),),
        in_specs=[
            pl.BlockSpec((n, c_in), lambda i: (0, 0)),
            pl.BlockSpec((n, be), lambda i: (0, i)),
            pl.BlockSpec((c_in, c_hid), lambda i: (0, 0)),
            pl.BlockSpec((1, c_hid), lambda i: (0, 0)),
        ],
        out_specs=[
            pl.BlockSpec((n, c_hid), lambda i: (0, 0)),
            pl.BlockSpec((be, c_hid), lambda i: (i, 0)),
            pl.BlockSpec((n, be), lambda i: (0, i)),
            pl.BlockSpec((n, 1), lambda i: (0, 0)),
        ],
        out_shape=[
            jax.ShapeDtypeStruct((n, c_hid), jnp.float32),
            jax.ShapeDtypeStruct((e, c_hid), jnp.float32),
            jax.ShapeDtypeStruct((n, e), jnp.int8),
            jax.ShapeDtypeStruct((n, 1), jnp.float32),
        ],
    )(x_0, incidence_1, wt, b2)

    bn = min(512, n)
    x0_out = pl.pallas_call(
        _pass_b,
        grid=(pl.cdiv(n, bn),),
        in_specs=[
            pl.BlockSpec((bn, e), lambda i: (i, 0)),
            pl.BlockSpec((e, c_hid), lambda i: (0, 0)),
            pl.BlockSpec((bn, c_hid), lambda i: (i, 0)),
            pl.BlockSpec((bn, 1), lambda i: (i, 0)),
        ],
        out_specs=pl.BlockSpec((bn, c_hid), lambda i: (i, 0)),
        out_shape=jax.ShapeDtypeStruct((n, c_hid), jnp.float32),
        scratch_shapes=[pltpu.VMEM((e, c_hid), jnp.bfloat16)],
    )(inc8, x_1, xlin, ns)

    return (x0_out, x_1)
